# baseline (speedup 1.0000x reference)
import functools

import jax
import jax.numpy as jnp
from jax import lax
from jax.experimental import pallas as pl
from jax.experimental.pallas import tpu as pltpu
from jax.experimental.pallas import tpu_sc as plsc

_DW = 128
_B = 4096 * 200
_NW = 32
_B_PER_W = _B // _NW
_CHUNK = 200
_T = _B_PER_W // _CHUNK
_NBUF = 4
_LAG = 3

_mesh = plsc.VectorSubcoreMesh(core_axis_name="c", subcore_axis_name="s")


@functools.partial(
    pl.kernel,
    mesh=_mesh,
    out_type=jax.ShapeDtypeStruct((_B, _DW), jnp.float32),
    scratch_types=(
        [pltpu.VMEM((_CHUNK,), jnp.int32) for _ in range(_NBUF)]
        + [pltpu.VMEM((_CHUNK, _DW), jnp.float32) for _ in range(_NBUF)]
        + [pltpu.SemaphoreType.DMA((_NBUF,)) for _ in range(3)]
    ),
    compiler_params=pltpu.CompilerParams(use_tc_tiling_on_sc=False),
)
def _gather_rows(idx_hbm, table_hbm, out_hbm,
                 i0, i1, i2, i3, r0, r1, r2, r3,
                 sem_i, sem_g, sem_o):
    wid = lax.axis_index("s") * 2 + lax.axis_index("c")
    base = wid * _B_PER_W
    idx_v = (i0, i1, i2, i3)
    rows_v = (r0, r1, r2, r3)

    def idx_copy(t, b):
        off = pl.multiple_of(base + t * _CHUNK, _CHUNK)
        return pltpu.make_async_copy(
            idx_hbm.at[pl.ds(off, _CHUNK)], idx_v[b], sem_i.at[b])

    def out_copy(t, b):
        off = pl.multiple_of(base + t * _CHUNK, _CHUNK)
        return pltpu.make_async_copy(
            rows_v[b].at[:, pl.ds(0, 64)],
            out_hbm.at[pl.ds(off, _CHUNK), pl.ds(0, 64)], sem_o.at[b])

    def gather(b):
        return pltpu.make_async_copy(
            table_hbm.at[idx_v[b]], rows_v[b], sem_g.at[b])

    for b in range(_NBUF):
        idx_copy(b, b).start()

    def body(i, carry):
        for b in range(_NBUF):
            t = _NBUF * i + b
            idx_copy(t, b).wait()

            @pl.when(i >= 1)
            def _(t=t, b=b):
                out_copy(t - _NBUF, b).wait()

            gather(b).start()

            td = t - _LAG
            b2 = (b + _NBUF - _LAG) % _NBUF
            if b < _LAG:
                @pl.when(i >= 1)
                def _(td=td, b2=b2):
                    gather(b2).wait()
                    out_copy(td, b2).start()
                    idx_copy(td + _NBUF, b2).start()
            else:
                gather(b2).wait()
                out_copy(td, b2).start()

                @pl.when(i < _T // _NBUF - 1)
                def _(td=td, b2=b2):
                    idx_copy(td + _NBUF, b2).start()
        return carry

    lax.fori_loop(0, _T // _NBUF, body, 0)

    for t in range(_T - _LAG, _T):
        b = t % _NBUF
        gather(b).wait()
        out_copy(t, b).start()
    for t in range(_T - _NBUF, _T):
        out_copy(t, t % _NBUF).wait()


def kernel(x, weight):
    idx = x.reshape(-1).astype(jnp.int32)
    wide = jnp.concatenate(
        [weight, jnp.zeros((weight.shape[0], _DW - weight.shape[1]),
                           weight.dtype)], axis=1)
    out = _gather_rows(idx, wide)
    return out[:, : weight.shape[1]].reshape(x.shape + (weight.shape[1],))


# speedup vs baseline: 1.0527x; 1.0527x over previous
import functools

import jax
import jax.numpy as jnp
from jax import lax
from jax.experimental import pallas as pl
from jax.experimental.pallas import tpu as pltpu
from jax.experimental.pallas import tpu_sc as plsc

_DW = 128
_B = 4096 * 200
_NW = 32
_B_PER_W = _B // _NW
_CHUNK = 200
_T = _B_PER_W // _CHUNK
_NBUF = 4
_LAG = 2

_mesh = plsc.VectorSubcoreMesh(core_axis_name="c", subcore_axis_name="s")


@functools.partial(
    pl.kernel,
    mesh=_mesh,
    out_type=jax.ShapeDtypeStruct((_B, _DW), jnp.float32),
    scratch_types=(
        [pltpu.VMEM((_CHUNK,), jnp.int32) for _ in range(_NBUF)]
        + [pltpu.VMEM((_CHUNK, _DW), jnp.float32) for _ in range(_NBUF)]
        + [pltpu.SemaphoreType.DMA((_NBUF,)) for _ in range(3)]
    ),
    compiler_params=pltpu.CompilerParams(use_tc_tiling_on_sc=False),
)
def _gather_rows(idx_hbm, table_hbm, out_hbm,
                 i0, i1, i2, i3, r0, r1, r2, r3,
                 sem_i, sem_g, sem_o):
    wid = lax.axis_index("s") * 2 + lax.axis_index("c")
    base = wid * _B_PER_W
    idx_v = (i0, i1, i2, i3)
    rows_v = (r0, r1, r2, r3)

    def idx_copy(t, b):
        off = pl.multiple_of(base + t * _CHUNK, _CHUNK)
        return pltpu.make_async_copy(
            idx_hbm.at[pl.ds(off, _CHUNK)], idx_v[b], sem_i.at[b])

    def out_copy(t, b):
        off = pl.multiple_of(base + t * _CHUNK, _CHUNK)
        return pltpu.make_async_copy(
            rows_v[b].at[:, pl.ds(0, 64)],
            out_hbm.at[pl.ds(off, _CHUNK), pl.ds(0, 64)], sem_o.at[b])

    def gather(b):
        return pltpu.make_async_copy(
            table_hbm.at[idx_v[b]], rows_v[b], sem_g.at[b])

    for b in range(_NBUF):
        idx_copy(b, b).start()

    def body(i, carry):
        for b in range(_NBUF):
            t = _NBUF * i + b
            idx_copy(t, b).wait()

            @pl.when(i >= 1)
            def _(t=t, b=b):
                out_copy(t - _NBUF, b).wait()

            gather(b).start()

            td = t - _LAG
            b2 = (b + _NBUF - _LAG) % _NBUF
            if b < _LAG:
                @pl.when(i >= 1)
                def _(td=td, b2=b2):
                    gather(b2).wait()
                    out_copy(td, b2).start()
                    idx_copy(td + _NBUF, b2).start()
            else:
                gather(b2).wait()
                out_copy(td, b2).start()

                @pl.when(i < _T // _NBUF - 1)
                def _(td=td, b2=b2):
                    idx_copy(td + _NBUF, b2).start()
        return carry

    lax.fori_loop(0, _T // _NBUF, body, 0)

    for t in range(_T - _LAG, _T):
        b = t % _NBUF
        gather(b).wait()
        out_copy(t, b).start()
    for t in range(_T - _NBUF, _T):
        out_copy(t, t % _NBUF).wait()


def kernel(x, weight):
    idx = x.reshape(-1).astype(jnp.int32)
    wide = jnp.concatenate(
        [weight, jnp.zeros((weight.shape[0], _DW - weight.shape[1]),
                           weight.dtype)], axis=1)
    out = _gather_rows(idx, wide)
    return out[:, : weight.shape[1]].reshape(x.shape + (weight.shape[1],))
